# baseline (speedup 1.0000x reference)
import functools

import jax
import jax.numpy as jnp
from jax import lax
from jax.experimental import pallas as pl
from jax.experimental.pallas import tpu as pltpu
from jax.experimental.pallas import tpu_sc as plsc

N = 10000
NP = 10240
E = 320000
D_IN = 128
H1 = 32
H2 = 64
HH = H2 // 2
G = 64
NC = 2
NS = 16
NW = NC * NS
CH = 128
NCH = 80
EPW = NCH * CH
EP = NW * EPW
NCH2 = EP // (NS * CH)
NBUF = 8
RPT = NP // NS
WPT = N // NS
BLK = 1000
NBLK = N // BLK

_mesh = plsc.VectorSubcoreMesh(
    core_axis_name="c", subcore_axis_name="s", num_cores=NC, num_subcores=NS)
_sc_params = pltpu.CompilerParams(use_tc_tiling_on_sc=False)


def _sc_degree(dst3, ones_rows, zeros8):
    @functools.partial(
        pl.kernel,
        out_type=jax.ShapeDtypeStruct((NC, N, 8), jnp.float32),
        mesh=_mesh,
        compiler_params=_sc_params,
        scratch_types=[
            pltpu.VMEM((NCH, CH), jnp.int32),
            pltpu.VMEM((CH, 8), jnp.float32),
            pltpu.VMEM_SHARED((NP, 8), jnp.float32),
        ],
    )
    def deg_kernel(dst_hbm, ones_hbm, z_hbm, out_hbm, didx, ones_v, acc):
        c = lax.axis_index("c")
        s = lax.axis_index("s")
        wid = c * NS + s
        r0 = s * RPT
        pltpu.sync_copy(z_hbm.at[pl.ds(r0, RPT)], acc.at[pl.ds(r0, RPT)])
        pltpu.sync_copy(dst_hbm.at[wid], didx)
        pltpu.sync_copy(ones_hbm, ones_v)
        plsc.subcore_barrier()

        def body(j, carry):
            pltpu.sync_copy(ones_v, acc.at[didx.at[j]], add=True)
            return carry

        lax.fori_loop(0, NCH, body, 0)
        plsc.subcore_barrier()
        w0 = s * WPT
        pltpu.sync_copy(acc.at[pl.ds(w0, WPT)], out_hbm.at[c, pl.ds(w0, WPT)])

    return deg_kernel(dst3, ones_rows, zeros8)


def _sc_aggregate(pk3, g2, zeros):
    @functools.partial(
        pl.kernel,
        out_type=jax.ShapeDtypeStruct((NC, N, HH), jnp.float32),
        mesh=_mesh,
        compiler_params=_sc_params,
        scratch_types=(
            [pltpu.VMEM((NCH2, CH), jnp.int32),
             pltpu.VMEM_SHARED((NP, HH), jnp.float32),
             pltpu.VMEM_SHARED((N, HH), jnp.float32)]
            + [pltpu.VMEM((CH, HH), jnp.float32) for _ in range(NBUF)]
            + [pltpu.VMEM((CH,), jnp.int32) for _ in range(NBUF)]
            + [pltpu.VMEM((CH,), jnp.int32) for _ in range(NBUF)]
            + [pltpu.SemaphoreType.DMA for _ in range(NBUF)]
        ),
    )
    def agg_kernel(pk_hbm, g_hbm, z_hbm, out_hbm, pidx, acc, gtab, *rest):
        rows = rest[:NBUF]
        sidx = rest[NBUF:2 * NBUF]
        didx = rest[2 * NBUF:3 * NBUF]
        gsem = rest[3 * NBUF:4 * NBUF]
        c = lax.axis_index("c")
        s = lax.axis_index("s")
        r0 = s * RPT
        w0 = s * WPT
        pltpu.sync_copy(g_hbm.at[c, pl.ds(w0, WPT)], gtab.at[pl.ds(w0, WPT)])
        pltpu.sync_copy(z_hbm.at[pl.ds(r0, RPT)], acc.at[pl.ds(r0, RPT)])
        pltpu.sync_copy(pk_hbm.at[s], pidx)
        plsc.subcore_barrier()

        def unpack(j, b):
            for k in range(CH // 16):
                v = pidx[j, pl.ds(k * 16, 16)]
                sidx[b][pl.ds(k * 16, 16)] = v >> 14
                didx[b][pl.ds(k * 16, 16)] = v & 0x3FFF

        for b in range(NBUF):
            unpack(b, b)
            pltpu.async_copy(gtab.at[sidx[b]], rows[b], gsem[b])

        def round_body(i, carry):
            j0 = i * NBUF
            for b in range(NBUF):
                pltpu.make_async_copy(gtab.at[sidx[b]], rows[b],
                                      gsem[b]).wait()
                pltpu.sync_copy(rows[b], acc.at[didx[b]], add=True)

                @pl.when(i < NCH2 // NBUF - 1)
                def _prefetch():
                    unpack(j0 + b + NBUF, b)
                    pltpu.async_copy(gtab.at[sidx[b]], rows[b], gsem[b])
            return carry

        lax.fori_loop(0, NCH2 // NBUF, round_body, 0)
        plsc.subcore_barrier()
        pltpu.sync_copy(acc.at[pl.ds(w0, WPT)], out_hbm.at[c, pl.ds(w0, WPT)])

    return agg_kernel(pk3, g2, zeros)


def _dinv_from(deg_ref):
    deg = deg_ref[0] + deg_ref[1]
    return lax.rsqrt(deg[:, 0:1] + 1.0)


def _split_g(h, dinv, g_ref, st_ref):
    gg = h * dinv
    g_ref[0] = gg[:, :HH]
    g_ref[1] = gg[:, HH:]
    st_ref[...] = h * (dinv * dinv)


def _tc_first(xp, W1m, degp):
    def body(x_ref, w_ref, deg_ref, g_ref, st_ref):
        dinv = _dinv_from(deg_ref)
        h = jnp.dot(x_ref[...], w_ref[...],
                    preferred_element_type=jnp.float32,
                    precision=lax.Precision.HIGHEST)
        _split_g(h, dinv, g_ref, st_ref)

    return pl.pallas_call(
        body,
        grid=(NBLK,),
        in_specs=[
            pl.BlockSpec((BLK, D_IN), lambda i: (i, 0)),
            pl.BlockSpec((D_IN, H2), lambda i: (0, 0)),
            pl.BlockSpec((NC, BLK, 8), lambda i: (0, i, 0)),
        ],
        out_specs=[
            pl.BlockSpec((NC, BLK, HH), lambda i: (0, i, 0)),
            pl.BlockSpec((BLK, H2), lambda i: (i, 0)),
        ],
        out_shape=[
            jax.ShapeDtypeStruct((NC, N, HH), jnp.float32),
            jax.ShapeDtypeStruct((N, H2), jnp.float32),
        ],
    )(xp, W1m, degp)


def _relu_layer(agg_ref, st_ref, b_ref, dinv):
    agg = jnp.concatenate([agg_ref[0], agg_ref[1]], axis=1)
    return jnp.maximum(agg * dinv + st_ref[...] + b_ref[...], 0.0)


def _tc_mid(aggp, st, degp, Wm, br):
    def body(agg_ref, st_ref, deg_ref, w_ref, b_ref, g_ref, sto_ref):
        dinv = _dinv_from(deg_ref)
        o = _relu_layer(agg_ref, st_ref, b_ref, dinv)
        h = jnp.dot(o, w_ref[...],
                    preferred_element_type=jnp.float32,
                    precision=lax.Precision.HIGHEST)
        _split_g(h, dinv, g_ref, sto_ref)

    return pl.pallas_call(
        body,
        grid=(NBLK,),
        in_specs=[
            pl.BlockSpec((NC, BLK, HH), lambda i: (0, i, 0)),
            pl.BlockSpec((BLK, H2), lambda i: (i, 0)),
            pl.BlockSpec((NC, BLK, 8), lambda i: (0, i, 0)),
            pl.BlockSpec((H2, H2), lambda i: (0, 0)),
            pl.BlockSpec((1, H2), lambda i: (0, 0)),
        ],
        out_specs=[
            pl.BlockSpec((NC, BLK, HH), lambda i: (0, i, 0)),
            pl.BlockSpec((BLK, H2), lambda i: (i, 0)),
        ],
        out_shape=[
            jax.ShapeDtypeStruct((NC, N, HH), jnp.float32),
            jax.ShapeDtypeStruct((N, H2), jnp.float32),
        ],
    )(aggp, st, degp, Wm, br)


def _tc_pool(aggp, st, degp, br, batp):
    def body(agg_ref, st_ref, deg_ref, b_ref, bat_ref, out_ref, sacc, cacc):
        i = pl.program_id(0)
        dinv = _dinv_from(deg_ref)
        h = _relu_layer(agg_ref, st_ref, b_ref, dinv)
        gids = lax.broadcasted_iota(jnp.int32, (BLK, G), 1)
        m = (bat_ref[...] == gids).astype(jnp.float32)
        part = lax.dot_general(m, h, (((0,), (0,)), ((), ())),
                               preferred_element_type=jnp.float32,
                               precision=lax.Precision.HIGHEST)
        cnt = lax.dot_general(m, jnp.ones((BLK, 8), jnp.float32),
                              (((0,), (0,)), ((), ())),
                              preferred_element_type=jnp.float32,
                              precision=lax.Precision.HIGHEST)

        @pl.when(i == 0)
        def _init():
            sacc[...] = part
            cacc[...] = cnt

        @pl.when(i > 0)
        def _accum():
            sacc[...] = sacc[...] + part
            cacc[...] = cacc[...] + cnt

        @pl.when(i == NBLK - 1)
        def _final():
            out_ref[...] = sacc[...] / jnp.maximum(cacc[...][:, 0:1], 1.0)

    return pl.pallas_call(
        body,
        grid=(NBLK,),
        in_specs=[
            pl.BlockSpec((NC, BLK, HH), lambda i: (0, i, 0)),
            pl.BlockSpec((BLK, H2), lambda i: (i, 0)),
            pl.BlockSpec((NC, BLK, 8), lambda i: (0, i, 0)),
            pl.BlockSpec((1, H2), lambda i: (0, 0)),
            pl.BlockSpec((BLK, 1), lambda i: (i, 0)),
        ],
        out_specs=pl.BlockSpec((G, H2), lambda i: (0, 0)),
        out_shape=jax.ShapeDtypeStruct((G, H2), jnp.float32),
        scratch_shapes=[
            pltpu.VMEM((G, H2), jnp.float32),
            pltpu.VMEM((G, 8), jnp.float32),
        ],
    )(aggp, st, degp, br, batp)


def kernel(x, edge_index, batch, W1, b1, W2, b2, W3, b3):
    ei = edge_index.astype(jnp.int32)
    srcp = jnp.concatenate([ei[0], jnp.zeros((EP - E,), jnp.int32)])
    dstp = jnp.concatenate([ei[1], jnp.full((EP - E,), NP - 1, jnp.int32)])
    pk3 = ((srcp << 14) | dstp).reshape(NS, NCH2, CH)
    dst3 = dstp.reshape(NW, NCH, CH)
    batp = batch.astype(jnp.int32).reshape(N, 1)
    ones_rows = jnp.ones((CH, 8), jnp.float32)
    z8 = jnp.zeros((NP, 8), jnp.float32)
    zh = jnp.zeros((NP, HH), jnp.float32)

    W1p = jnp.zeros((D_IN, H2), jnp.float32).at[:, :H1].set(W1)
    W2p = jnp.zeros((H2, H2), jnp.float32).at[:H1, :].set(W2)
    b1p = jnp.zeros((1, H2), jnp.float32).at[0, :H1].set(b1)
    b2r = b2.reshape(1, H2)
    b3r = b3.reshape(1, H2)

    degp = _sc_degree(dst3, ones_rows, z8)
    g1, st1 = _tc_first(x, W1p, degp)
    agg1 = _sc_aggregate(pk3, g1, zh)
    g2, st2 = _tc_mid(agg1, st1, degp, W2p, b1p)
    agg2 = _sc_aggregate(pk3, g2, zh)
    g3, st3 = _tc_mid(agg2, st2, degp, W3, b2r)
    agg3 = _sc_aggregate(pk3, g3, zh)
    return _tc_pool(agg3, st3, degp, b3r, batp)


# speedup vs baseline: 1.1255x; 1.1255x over previous
import functools

import jax
import jax.numpy as jnp
from jax import lax
from jax.experimental import pallas as pl
from jax.experimental.pallas import tpu as pltpu
from jax.experimental.pallas import tpu_sc as plsc

N = 10000
NP = 10240
E = 320000
D_IN = 128
H1 = 32
H2 = 64
HH = H2 // 2
G = 64
NC = 2
NS = 16
NW = NC * NS
CH = 128
NCH = 80
EPW = NCH * CH
EP = NW * EPW
NCH2 = EP // (NS * CH)
NBUF = 8
RPT = NP // NS
WPT = N // NS
ZR = 80
BLK = 1000
NBLK = N // BLK

_mesh = plsc.VectorSubcoreMesh(
    core_axis_name="c", subcore_axis_name="s", num_cores=NC, num_subcores=NS)
_sc_params = pltpu.CompilerParams(use_tc_tiling_on_sc=False)


def _fill(ref, rows, cols, value):
    v = jnp.full((16,), value, jnp.float32)
    for r in range(rows):
        for k in range(cols // 16):
            ref[r, pl.ds(k * 16, 16)] = v


def _sc_degree(dst3, ones_rows, zeros8):
    @functools.partial(
        pl.kernel,
        out_type=jax.ShapeDtypeStruct((NC, N, 8), jnp.float32),
        mesh=_mesh,
        compiler_params=_sc_params,
        scratch_types=[
            pltpu.VMEM((NCH, CH), jnp.int32),
            pltpu.VMEM((CH, 8), jnp.float32),
            pltpu.VMEM_SHARED((NP, 8), jnp.float32),
        ],
    )
    def deg_kernel(dst_hbm, ones_hbm, z_hbm, out_hbm, didx, ones_v, acc):
        c = lax.axis_index("c")
        s = lax.axis_index("s")
        wid = c * NS + s
        r0 = s * RPT
        pltpu.sync_copy(z_hbm.at[pl.ds(r0, RPT)], acc.at[pl.ds(r0, RPT)])
        pltpu.sync_copy(dst_hbm.at[wid], didx)
        pltpu.sync_copy(ones_hbm, ones_v)
        plsc.subcore_barrier()

        def body(j, carry):
            pltpu.sync_copy(ones_v, acc.at[didx.at[j]], add=True)
            return carry

        lax.fori_loop(0, NCH, body, 0)
        plsc.subcore_barrier()
        w0 = s * WPT
        pltpu.sync_copy(acc.at[pl.ds(w0, WPT)], out_hbm.at[c, pl.ds(w0, WPT)])

    return deg_kernel(dst3, ones_rows, zeros8)


def _sc_aggregate(pk3, gst):
    @functools.partial(
        pl.kernel,
        out_type=jax.ShapeDtypeStruct((N, 128), jnp.float32),
        mesh=_mesh,
        compiler_params=_sc_params,
        scratch_types=(
            [pltpu.VMEM((NCH2, CH), jnp.int32),
             pltpu.VMEM((ZR, HH), jnp.float32),
             pltpu.VMEM_SHARED((NP, HH), jnp.float32),
             pltpu.VMEM_SHARED((N, HH), jnp.float32)]
            + [pltpu.VMEM((CH, HH), jnp.float32) for _ in range(NBUF)]
            + [pltpu.VMEM((CH,), jnp.int32) for _ in range(NBUF)]
            + [pltpu.VMEM((CH,), jnp.int32) for _ in range(NBUF)]
            + [pltpu.SemaphoreType.DMA for _ in range(NBUF)]
        ),
    )
    def agg_kernel(pk_hbm, g_hbm, out_hbm, pidx, zbuf, acc, gtab, *rest):
        rows = rest[:NBUF]
        sidx = rest[NBUF:2 * NBUF]
        didx = rest[2 * NBUF:3 * NBUF]
        gsem = rest[3 * NBUF:4 * NBUF]
        c = lax.axis_index("c")
        s = lax.axis_index("s")
        r0 = s * RPT
        w0 = s * WPT
        @pl.when(c == 0)
        def _stage0():
            pltpu.sync_copy(g_hbm.at[pl.ds(w0, WPT), pl.ds(0, HH)],
                            gtab.at[pl.ds(w0, WPT)])

        @pl.when(c == 1)
        def _stage1():
            pltpu.sync_copy(g_hbm.at[pl.ds(w0, WPT), pl.ds(HH, HH)],
                            gtab.at[pl.ds(w0, WPT)])
        _fill(zbuf, ZR, HH, 0.0)
        for q in range(RPT // ZR):
            pltpu.sync_copy(zbuf, acc.at[pl.ds(r0 + q * ZR, ZR)])
        pltpu.sync_copy(pk_hbm.at[s], pidx)
        plsc.subcore_barrier()

        def unpack(j, b):
            for k in range(CH // 16):
                v = pidx[j, pl.ds(k * 16, 16)]
                sidx[b][pl.ds(k * 16, 16)] = v >> 14
                didx[b][pl.ds(k * 16, 16)] = v & 0x3FFF

        for b in range(NBUF):
            unpack(b, b)
            pltpu.async_copy(gtab.at[sidx[b]], rows[b], gsem[b])

        def round_body(i, carry):
            j0 = i * NBUF
            for b in range(NBUF):
                pltpu.make_async_copy(gtab.at[sidx[b]], rows[b],
                                      gsem[b]).wait()
                pltpu.sync_copy(rows[b], acc.at[didx[b]], add=True)

                @pl.when(i < NCH2 // NBUF - 1)
                def _prefetch():
                    unpack(j0 + b + NBUF, b)
                    pltpu.async_copy(gtab.at[sidx[b]], rows[b], gsem[b])
            return carry

        lax.fori_loop(0, NCH2 // NBUF, round_body, 0)
        plsc.subcore_barrier()

        @pl.when(c == 0)
        def _write0():
            pltpu.sync_copy(acc.at[pl.ds(w0, WPT)],
                            out_hbm.at[pl.ds(w0, WPT), pl.ds(0, HH)])

        @pl.when(c == 1)
        def _write1():
            pltpu.sync_copy(acc.at[pl.ds(w0, WPT)],
                            out_hbm.at[pl.ds(w0, WPT), pl.ds(HH, HH)])

    return agg_kernel(pk3, gst)


def _dinv_from(deg_ref):
    deg = deg_ref[0] + deg_ref[1]
    return lax.rsqrt(deg[:, 0:1] + 1.0)


def _gst_of(h, dinv):
    return jnp.concatenate([h * dinv, h * (dinv * dinv)], axis=1)


def _tc_first(x, W1m, degp):
    def body(x_ref, w_ref, deg_ref, gst_ref):
        dinv = _dinv_from(deg_ref)
        h = jnp.dot(x_ref[...], w_ref[...],
                    preferred_element_type=jnp.float32,
                    precision=lax.Precision.HIGHEST)
        gst_ref[...] = _gst_of(h, dinv)

    return pl.pallas_call(
        body,
        grid=(NBLK,),
        in_specs=[
            pl.BlockSpec((BLK, D_IN), lambda i: (i, 0)),
            pl.BlockSpec((D_IN, H2), lambda i: (0, 0)),
            pl.BlockSpec((NC, BLK, 8), lambda i: (0, i, 0)),
        ],
        out_specs=pl.BlockSpec((BLK, 128), lambda i: (i, 0)),
        out_shape=jax.ShapeDtypeStruct((N, 128), jnp.float32),
    )(x, W1m, degp)


def _relu_layer(agg_ref, gst_ref, b_ref, dinv):
    agg = agg_ref[:, :H2]
    st = gst_ref[:, H2:]
    return jnp.maximum(agg * dinv + st + b_ref[...], 0.0)


def _tc_mid(agg, gst, degp, Wm, br):
    def body(agg_ref, gst_in_ref, deg_ref, w_ref, b_ref, gst_ref):
        dinv = _dinv_from(deg_ref)
        o = _relu_layer(agg_ref[...], gst_in_ref[...], b_ref, dinv)
        h = jnp.dot(o, w_ref[...],
                    preferred_element_type=jnp.float32,
                    precision=lax.Precision.HIGHEST)
        gst_ref[...] = _gst_of(h, dinv)

    return pl.pallas_call(
        body,
        grid=(NBLK,),
        in_specs=[
            pl.BlockSpec((BLK, 128), lambda i: (i, 0)),
            pl.BlockSpec((BLK, 128), lambda i: (i, 0)),
            pl.BlockSpec((NC, BLK, 8), lambda i: (0, i, 0)),
            pl.BlockSpec((H2, H2), lambda i: (0, 0)),
            pl.BlockSpec((1, H2), lambda i: (0, 0)),
        ],
        out_specs=pl.BlockSpec((BLK, 128), lambda i: (i, 0)),
        out_shape=jax.ShapeDtypeStruct((N, 128), jnp.float32),
    )(agg, gst, degp, Wm, br)


def _tc_pool(agg, gst, degp, br, batp):
    def body(agg_ref, gst_ref, deg_ref, b_ref, bat_ref, out_ref, sacc, cacc):
        i = pl.program_id(0)
        dinv = _dinv_from(deg_ref)
        h = _relu_layer(agg_ref[...], gst_ref[...], b_ref, dinv)
        gids = lax.broadcasted_iota(jnp.int32, (BLK, G), 1)
        m = (bat_ref[...] == gids).astype(jnp.float32)
        part = lax.dot_general(m, h, (((0,), (0,)), ((), ())),
                               preferred_element_type=jnp.float32,
                               precision=lax.Precision.HIGHEST)
        cnt = lax.dot_general(m, jnp.ones((BLK, 8), jnp.float32),
                              (((0,), (0,)), ((), ())),
                              preferred_element_type=jnp.float32,
                              precision=lax.Precision.HIGHEST)

        @pl.when(i == 0)
        def _init():
            sacc[...] = part
            cacc[...] = cnt

        @pl.when(i > 0)
        def _accum():
            sacc[...] = sacc[...] + part
            cacc[...] = cacc[...] + cnt

        @pl.when(i == NBLK - 1)
        def _final():
            out_ref[...] = sacc[...] / jnp.maximum(cacc[...][:, 0:1], 1.0)

    return pl.pallas_call(
        body,
        grid=(NBLK,),
        in_specs=[
            pl.BlockSpec((BLK, 128), lambda i: (i, 0)),
            pl.BlockSpec((BLK, 128), lambda i: (i, 0)),
            pl.BlockSpec((NC, BLK, 8), lambda i: (0, i, 0)),
            pl.BlockSpec((1, H2), lambda i: (0, 0)),
            pl.BlockSpec((BLK, 1), lambda i: (i, 0)),
        ],
        out_specs=pl.BlockSpec((G, H2), lambda i: (0, 0)),
        out_shape=jax.ShapeDtypeStruct((G, H2), jnp.float32),
        scratch_shapes=[
            pltpu.VMEM((G, H2), jnp.float32),
            pltpu.VMEM((G, 8), jnp.float32),
        ],
    )(agg, gst, degp, br, batp)


def kernel(x, edge_index, batch, W1, b1, W2, b2, W3, b3):
    ei = edge_index.astype(jnp.int32)
    srcp = jnp.concatenate([ei[0], jnp.zeros((EP - E,), jnp.int32)])
    dstp = jnp.concatenate([ei[1], jnp.full((EP - E,), NP - 1, jnp.int32)])
    pk3 = ((srcp << 14) | dstp).reshape(NS, NCH2, CH)
    dst3 = dstp.reshape(NW, NCH, CH)
    batp = batch.astype(jnp.int32).reshape(N, 1)
    ones_rows = jnp.ones((CH, 8), jnp.float32)
    z8 = jnp.zeros((NP, 8), jnp.float32)

    W1p = jnp.zeros((D_IN, H2), jnp.float32).at[:, :H1].set(W1)
    W2p = jnp.zeros((H2, H2), jnp.float32).at[:H1, :].set(W2)
    b1p = jnp.zeros((1, H2), jnp.float32).at[0, :H1].set(b1)
    b2r = b2.reshape(1, H2)
    b3r = b3.reshape(1, H2)

    degp = _sc_degree(dst3, ones_rows, z8)
    gst1 = _tc_first(x, W1p, degp)
    agg1 = _sc_aggregate(pk3, gst1)
    gst2 = _tc_mid(agg1, gst1, degp, W2p, b1p)
    agg2 = _sc_aggregate(pk3, gst2)
    gst3 = _tc_mid(agg2, gst2, degp, W3, b2r)
    agg3 = _sc_aggregate(pk3, gst3)
    return _tc_pool(agg3, gst3, degp, b3r, batp)
